# baseline (speedup 1.0000x reference)
import functools

import jax
import jax.numpy as jnp
from jax import lax
from jax.experimental import pallas as pl
from jax.experimental.pallas import tpu as pltpu
from jax.experimental.pallas import tpu_sc as plsc

N_NODES = 100000
DIM = 16
NC = 2
NS = 16
NW = NC * NS
CHUNK = 512
N_ACC = 102400
ROWS_PER_TILE = N_ACC // NS
DUMMY_ROW = N_NODES


def _sc_scatter_pass(table, edge_index, tail_src, tail_dst, n_main, n_per_w):
    mesh = plsc.VectorSubcoreMesh(core_axis_name="c", subcore_axis_name="s")
    have_tail = tail_src is not None

    @functools.partial(
        pl.kernel,
        out_type=jax.ShapeDtypeStruct((NC, N_ACC, DIM), jnp.float32),
        mesh=mesh,
        scratch_types=[
            pltpu.VMEM_SHARED((N_ACC, DIM), jnp.float32),
            pltpu.VMEM((2, CHUNK), jnp.int32),
            pltpu.VMEM((2, CHUNK), jnp.int32),
            pltpu.VMEM((2, CHUNK, DIM), jnp.float32),
            pltpu.SemaphoreType.DMA((2,)),
            pltpu.SemaphoreType.DMA((2,)),
        ],
        compiler_params=pltpu.CompilerParams(use_tc_tiling_on_sc=False),
    )
    def body(table_hbm, edge_hbm, tsrc_hbm, tdst_hbm, out_hbm,
             acc, sbuf, dbuf, rows, gsem, ssem):
        cid = lax.axis_index("c")
        sid = lax.axis_index("s")
        wid = sid * NC + cid

        for i in range(128):
            rows[0, i, :] = jnp.zeros((DIM,), jnp.float32)
        base = sid * ROWS_PER_TILE
        for j in range(ROWS_PER_TILE // 128):
            pltpu.sync_copy(
                rows.at[0, pl.ds(0, 128)], acc.at[pl.ds(base + j * 128, 128)]
            )
        plsc.subcore_barrier()

        def stage_and_fire(sc, slot):
            off = wid * n_per_w + sc * CHUNK
            pltpu.sync_copy(edge_hbm.at[0, pl.ds(off, CHUNK)], sbuf.at[slot])
            pltpu.sync_copy(edge_hbm.at[1, pl.ds(off, CHUNK)], dbuf.at[slot])
            pltpu.async_copy(table_hbm.at[sbuf.at[slot]], rows.at[slot], gsem.at[slot])

        stage_and_fire(0, 0)

        def superchunk(sc, carry):
            p = lax.rem(sc, 2)
            q = 1 - p

            @pl.when(sc + 1 < n_main)
            def _():
                stage_and_fire(sc + 1, q)

            pltpu.make_async_copy(
                table_hbm.at[sbuf.at[p]], rows.at[p], gsem.at[p]
            ).wait()
            pltpu.async_copy(
                rows.at[p], acc.at[dbuf.at[p]], ssem.at[p], add=True
            ).wait()
            return carry

        lax.fori_loop(0, n_main, superchunk, 0)

        if have_tail:
            pltpu.sync_copy(tsrc_hbm.at[wid], sbuf.at[0])
            pltpu.sync_copy(tdst_hbm.at[wid], dbuf.at[0])
            pltpu.async_copy(table_hbm.at[sbuf.at[0]], rows.at[0], gsem.at[0]).wait()
            pltpu.async_copy(rows.at[0], acc.at[dbuf.at[0]], ssem.at[0], add=True).wait()

        plsc.subcore_barrier()
        pltpu.sync_copy(
            acc.at[pl.ds(base, ROWS_PER_TILE)],
            out_hbm.at[cid, pl.ds(base, ROWS_PER_TILE)],
        )

    if not have_tail:
        tail_src = jnp.zeros((NW, 8), jnp.int32)
        tail_dst = jnp.full((NW, 8), DUMMY_ROW, jnp.int32)
    return body(table, edge_index, tail_src, tail_dst)


def _tc_affine(partials, w, b, relu):
    blk = 4096

    def body(p_ref, w_ref, b_ref, o_ref):
        p = p_ref[...]
        z = jnp.dot(p[0] + p[1], w_ref[...], preferred_element_type=jnp.float32)
        z = z + b_ref[...]
        o_ref[...] = jnp.maximum(z, 0.0) if relu else z

    return pl.pallas_call(
        body,
        grid=(N_ACC // blk,),
        in_specs=[
            pl.BlockSpec((NC, blk, DIM), lambda i: (0, i, 0)),
            pl.BlockSpec((DIM, DIM), lambda i: (0, 0)),
            pl.BlockSpec((1, DIM), lambda i: (0, 0)),
        ],
        out_specs=pl.BlockSpec((blk, DIM), lambda i: (i, 0)),
        out_shape=jax.ShapeDtypeStruct((N_ACC, DIM), jnp.float32),
    )(partials, w, b.reshape(1, DIM))


def kernel(entity_emb, W1, b1, W2, b2, edge_index):
    n_edges = edge_index.shape[1]
    assert n_edges % NW == 0
    n_per_w = n_edges // NW
    n_main = n_per_w // CHUNK
    tail_len = n_per_w - n_main * CHUNK

    if tail_len:
        ei = edge_index.reshape(2, NW, n_per_w)
        tail = ei[:, :, n_main * CHUNK:]
        npad = CHUNK - tail_len
        pad_dst = (
            DUMMY_ROW
            + jnp.arange(NW * npad, dtype=jnp.int32) % (N_ACC - N_NODES)
        ).reshape(NW, npad)
        tail_src = jnp.concatenate(
            [tail[0], jnp.zeros((NW, npad), jnp.int32)], axis=1)
        tail_dst = jnp.concatenate([tail[1], pad_dst], axis=1)
    else:
        tail_src = tail_dst = None

    p1 = _sc_scatter_pass(entity_emb, edge_index, tail_src, tail_dst, n_main, n_per_w)
    x = _tc_affine(p1, W1, b1, relu=True)
    p2 = _sc_scatter_pass(x, edge_index, tail_src, tail_dst, n_main, n_per_w)
    out = _tc_affine(p2, W2, b2, relu=False)
    return out[:N_NODES]
